# Initial kernel scaffold; baseline (speedup 1.0000x reference)
import jax
import jax.numpy as jnp
from jax.experimental import pallas as pl


def kernel(x, edge_index, pos, params):
    raise NotImplementedError("write your pallas kernel here")


# speedup vs baseline: 6.5689x; 6.5689x over previous
import functools
import jax
import jax.numpy as jnp
from jax import lax
from jax.experimental import pallas as pl
from jax.experimental.pallas import tpu as pltpu
from jax.experimental.pallas import tpu_sc as plsc

_NC = 2
_NS = 16
_NW = _NC * _NS
_L = 16

_C = 128
_K = 80
_DPAD = 16


def _sc_edge_scatter(p, q, srcw, dstw):
    n = p.shape[0]
    nch = srcw.shape[1]
    rpt = n // _NS
    assert srcw.shape == (_NW, nch, _K)
    nz, zrem = rpt // _K, rpt % _K

    mesh = plsc.VectorSubcoreMesh(core_axis_name="c", subcore_axis_name="s")

    @functools.partial(
        pl.kernel,
        out_type=[
            jax.ShapeDtypeStruct((_NC, n, _C), jnp.float32),
            jax.ShapeDtypeStruct((_NC, n, _DPAD), jnp.float32),
        ],
        mesh=mesh,
        compiler_params=pltpu.CompilerParams(use_tc_tiling_on_sc=False),
        scratch_types=[
            pltpu.VMEM((1, _K), jnp.int32),
            pltpu.VMEM((1, _K), jnp.int32),
            pltpu.VMEM((_K, _C), jnp.float32),
            pltpu.VMEM((_K, _C), jnp.float32),
            pltpu.VMEM((_K, _C), jnp.float32),
            pltpu.VMEM((_K, _DPAD), jnp.float32),
            pltpu.VMEM((_K, _DPAD), jnp.float32),
            pltpu.VMEM_SHARED((n, _C), jnp.float32),
            pltpu.VMEM_SHARED((n, _DPAD), jnp.float32),
            pltpu.SemaphoreType.DMA,
            pltpu.SemaphoreType.DMA,
        ],
    )
    def k(p_hbm, q_hbm, src_hbm, dst_hbm, s_out, d_out,
          sidx, didx, pbuf, qbuf, rbuf, dzbuf, obuf,
          s_sh, d_sh, semp, semq):
        cid = lax.axis_index("c")
        sid = lax.axis_index("s")
        wid = cid * _NS + sid

        zeros = jnp.zeros((_L,), jnp.float32)
        ones = jnp.ones((_L,), jnp.float32)

        def fill_z(r, _):
            for c in range(_C // _L):
                rbuf[r, pl.ds(c * _L, _L)] = zeros
            dzbuf[r, :] = zeros
            obuf[r, :] = ones
            return 0
        lax.fori_loop(0, _K, fill_z, 0)

        base = sid * rpt
        for t in range(nz):
            pltpu.sync_copy(rbuf, s_sh.at[pl.ds(base + t * _K, _K)])
            pltpu.sync_copy(dzbuf, d_sh.at[pl.ds(base + t * _K, _K)])
        if zrem:
            pltpu.sync_copy(rbuf.at[pl.ds(0, zrem)],
                            s_sh.at[pl.ds(base + nz * _K, zrem)])
            pltpu.sync_copy(dzbuf.at[pl.ds(0, zrem)],
                            d_sh.at[pl.ds(base + nz * _K, zrem)])

        plsc.subcore_barrier()

        def chunk(j, _):
            pltpu.sync_copy(src_hbm.at[wid, pl.ds(j, 1)], sidx)
            pltpu.sync_copy(dst_hbm.at[wid, pl.ds(j, 1)], didx)
            cp = pltpu.async_copy(p_hbm.at[sidx.at[0]], pbuf, semp)
            cq = pltpu.async_copy(q_hbm.at[didx.at[0]], qbuf, semq)
            cp.wait()
            cq.wait()

            def row(r, _):
                for c in range(_C // _L):
                    sl = pl.ds(c * _L, _L)
                    rbuf[r, sl] = jnp.maximum(pbuf[r, sl] + qbuf[r, sl], 0.0)
                return 0
            lax.fori_loop(0, _K, row, 0)

            pltpu.sync_copy(rbuf, s_sh.at[didx.at[0]], add=True)
            pltpu.sync_copy(obuf, d_sh.at[didx.at[0]], add=True)
            return 0
        lax.fori_loop(0, nch, chunk, 0)

        plsc.subcore_barrier()

        pltpu.sync_copy(s_sh.at[pl.ds(base, rpt)], s_out.at[cid, pl.ds(base, rpt)])
        pltpu.sync_copy(d_sh.at[pl.ds(base, rpt)], d_out.at[cid, pl.ds(base, rpt)])

    return k(p, q, srcw, dstw)


_BR = 2000


def _row_spec(width):
    return pl.BlockSpec((_BR, width), lambda i: (i, 0))


def _w_spec(shape):
    return pl.BlockSpec(shape, lambda i: (0,) * len(shape))


def _enc_pq_body(x, pos, w1, b1, w2, b2, wp, wq, wd, bq, h_o, p_o, q_o):
    h = jnp.maximum(x[...] @ w1[...] + b1[...], 0.0) @ w2[...] + b2[...]
    pd = pos[...] @ wd[...]
    h_o[...] = h
    p_o[...] = h @ wp[...] + pd
    q_o[...] = h @ wq[...] - pd + bq[...]


def _enc_pq(x, pos8, w1t, b1, w2t, b2, wpt, wqt, wdt, bq):
    n = x.shape[0]
    grid = (n // _BR,)
    out = [jax.ShapeDtypeStruct((n, _C), jnp.float32)] * 3
    return pl.pallas_call(
        _enc_pq_body,
        grid=grid,
        in_specs=[
            _row_spec(_C), _row_spec(8),
            _w_spec((_C, _C)), _w_spec((1, _C)), _w_spec((_C, _C)), _w_spec((1, _C)),
            _w_spec((_C, _C)), _w_spec((_C, _C)), _w_spec((8, _C)), _w_spec((1, _C)),
        ],
        out_specs=[_row_spec(_C)] * 3,
        out_shape=out,
    )(x, pos8, w1t, b1, w2t, b2, wpt, wqt, wdt, bq)


def _node_core(h, s0, s1, deg, w2et, b2e, wn1at, wn1bt, bn1, wn2t, bn2):
    s = s0[...] + s1[...]
    agg = s @ w2et[...] + deg[...] * b2e[...]
    z = h[...] @ wn1at[...] + agg @ wn1bt[...] + bn1[...]
    return h[...] + jnp.maximum(z, 0.0) @ wn2t[...] + bn2[...]


def _node_pq_body(h, s0, s1, deg, pos,
                  w2et, b2e, wn1at, wn1bt, bn1, wn2t, bn2,
                  wp, wq, wd, bq, h_o, p_o, q_o):
    hn = _node_core(h, s0, s1, deg, w2et, b2e, wn1at, wn1bt, bn1, wn2t, bn2)
    pd = pos[...] @ wd[...]
    h_o[...] = hn
    p_o[...] = hn @ wp[...] + pd
    q_o[...] = hn @ wq[...] - pd + bq[...]


def _node_pq(h, s0, s1, deg, pos8, w2et, b2e, wn1at, wn1bt, bn1, wn2t, bn2,
             wpt, wqt, wdt, bq):
    n = h.shape[0]
    grid = (n // _BR,)
    out = [jax.ShapeDtypeStruct((n, _C), jnp.float32)] * 3
    return pl.pallas_call(
        _node_pq_body,
        grid=grid,
        in_specs=[
            _row_spec(_C), _row_spec(_C), _row_spec(_C), _row_spec(1), _row_spec(8),
            _w_spec((_C, _C)), _w_spec((1, _C)),
            _w_spec((_C, _C)), _w_spec((_C, _C)), _w_spec((1, _C)),
            _w_spec((_C, _C)), _w_spec((1, _C)),
            _w_spec((_C, _C)), _w_spec((_C, _C)), _w_spec((8, _C)), _w_spec((1, _C)),
        ],
        out_specs=[_row_spec(_C)] * 3,
        out_shape=out,
    )(h, s0, s1, deg, pos8, w2et, b2e, wn1at, wn1bt, bn1, wn2t, bn2,
      wpt, wqt, wdt, bq)


def _node_dec_body(h, s0, s1, deg,
                   w2et, b2e, wn1at, wn1bt, bn1, wn2t, bn2,
                   wd1, bd1, wd2, bd2, y_o):
    hn = _node_core(h, s0, s1, deg, w2et, b2e, wn1at, wn1bt, bn1, wn2t, bn2)
    y_o[...] = jnp.maximum(hn @ wd1[...] + bd1[...], 0.0) @ wd2[...] + bd2[...]


def _node_dec(h, s0, s1, deg, w2et, b2e, wn1at, wn1bt, bn1, wn2t, bn2,
              wd1t, bd1, wd2t, bd2):
    n = h.shape[0]
    grid = (n // _BR,)
    return pl.pallas_call(
        _node_dec_body,
        grid=grid,
        in_specs=[
            _row_spec(_C), _row_spec(_C), _row_spec(_C), _row_spec(1),
            _w_spec((_C, _C)), _w_spec((1, _C)),
            _w_spec((_C, _C)), _w_spec((_C, _C)), _w_spec((1, _C)),
            _w_spec((_C, _C)), _w_spec((1, _C)),
            _w_spec((_C, _C)), _w_spec((1, _C)), _w_spec((_C, _C)), _w_spec((1, _C)),
        ],
        out_specs=[_row_spec(_C)],
        out_shape=[jax.ShapeDtypeStruct((n, _C), jnp.float32)],
    )(h, s0, s1, deg, w2et, b2e, wn1at, wn1bt, bn1, wn2t, bn2,
      wd1t, bd1, wd2t, bd2)[0]


def _row(b):
    return b.reshape(1, -1)


@jax.jit
def kernel(x, edge_index, pos, params):
    n, c = x.shape
    e = edge_index.shape[1]
    epw = e // _NW
    nch = epw // _K
    assert nch * _K == epw and c == _C

    enc, dec, mps = params["enc"], params["dec"], params["mp"]
    pos8 = jnp.pad(pos, ((0, 0), (0, 5)))

    def edge_split(mp):
        w1 = mp["edge"][0]["W"]
        a, b, cm, d = w1[:, :c], w1[:, c:2*c], w1[:, 2*c:3*c], w1[:, 3*c:]
        wpt = (a + cm).T
        wqt = (b - cm).T
        wdt = jnp.pad(d, ((0, 0), (0, 5))).T
        return wpt, wqt, wdt, _row(mp["edge"][0]["b"])

    def node_split(mp):
        wn1 = mp["node"][0]["W"]
        return (mp["edge"][1]["W"].T, _row(mp["edge"][1]["b"]),
                wn1[:, :c].T, wn1[:, c:].T, _row(mp["node"][0]["b"]),
                mp["node"][1]["W"].T, _row(mp["node"][1]["b"]))

    src = edge_index[0].astype(jnp.int32).reshape(_NW, nch, _K)
    dst = edge_index[1].astype(jnp.int32).reshape(_NW, nch, _K)

    wpt, wqt, wdt, bq = edge_split(mps[0])
    h, p1, q1 = _enc_pq(x, pos8, enc[0]["W"].T, _row(enc[0]["b"]),
                        enc[1]["W"].T, _row(enc[1]["b"]), wpt, wqt, wdt, bq)

    s_part, d_part = _sc_edge_scatter(p1, q1, src, dst)
    deg = (d_part[0, :, 0] + d_part[1, :, 0]).reshape(n, 1)

    wpt2, wqt2, wdt2, bq2 = edge_split(mps[1])
    h2, p2, q2 = _node_pq(h, s_part[0], s_part[1], deg, pos8,
                          *node_split(mps[0]), wpt2, wqt2, wdt2, bq2)

    s_part2, _ = _sc_edge_scatter(p2, q2, src, dst)

    return _node_dec(h2, s_part2[0], s_part2[1], deg,
                     *node_split(mps[1]),
                     dec[0]["W"].T, _row(dec[0]["b"]),
                     dec[1]["W"].T, _row(dec[1]["b"]))


# speedup vs baseline: 11.9202x; 1.8146x over previous
import functools
import jax
import jax.numpy as jnp
from jax import lax
from jax.experimental import pallas as pl
from jax.experimental.pallas import tpu as pltpu
from jax.experimental.pallas import tpu_sc as plsc

_NC = 2
_NS = 16
_NW = _NC * _NS
_L = 16

_C = 128
_K = 40
_DPAD = 16


def _sc_edge_scatter(p, q, srcw, dstw):
    n = p.shape[0]
    nch = srcw.shape[1]
    rpt = n // _NS
    assert srcw.shape == (_NW, nch, _K) and nch % 2 == 0
    nz, zrem = rpt // _K, rpt % _K

    mesh = plsc.VectorSubcoreMesh(core_axis_name="c", subcore_axis_name="s")

    @functools.partial(
        pl.kernel,
        out_type=jax.ShapeDtypeStruct((_NC, n, _C), jnp.float32),
        mesh=mesh,
        compiler_params=pltpu.CompilerParams(use_tc_tiling_on_sc=False),
        scratch_types=[
            pltpu.VMEM((nch, _K), jnp.int32),
            pltpu.VMEM((nch, _K), jnp.int32),
            [pltpu.VMEM((_K, _C), jnp.float32)] * 2,
            [pltpu.VMEM((_K, _C), jnp.float32)] * 2,
            pltpu.VMEM_SHARED((n, _C), jnp.float32),
            [pltpu.SemaphoreType.DMA] * 2,
            [pltpu.SemaphoreType.DMA] * 2,
            [pltpu.SemaphoreType.DMA] * 2,
        ],
    )
    def k(p_hbm, q_hbm, src_hbm, dst_hbm, s_out,
          sidx, didx, pb, qb, s_sh, gsp, gsq, ssem):
        cid = lax.axis_index("c")
        sid = lax.axis_index("s")
        wid = cid * _NS + sid

        pltpu.sync_copy(src_hbm.at[wid], sidx)
        pltpu.sync_copy(dst_hbm.at[wid], didx)

        zeros = jnp.zeros((_L,), jnp.float32)

        def fill_z(r, _):
            for c in range(_C // _L):
                pb[0][r, pl.ds(c * _L, _L)] = zeros
            return 0
        lax.fori_loop(0, _K, fill_z, 0)
        base = sid * rpt
        for t in range(nz):
            pltpu.sync_copy(pb[0], s_sh.at[pl.ds(base + t * _K, _K)])
        if zrem:
            pltpu.sync_copy(pb[0].at[pl.ds(0, zrem)],
                            s_sh.at[pl.ds(base + nz * _K, zrem)])

        plsc.subcore_barrier()

        def start_gather(j, par):
            pltpu.async_copy(p_hbm.at[sidx.at[j]], pb[par], gsp[par])
            pltpu.async_copy(q_hbm.at[didx.at[j]], qb[par], gsq[par])

        def wait_gather(j, par):
            pltpu.make_async_copy(p_hbm.at[sidx.at[j]], pb[par], gsp[par]).wait()
            pltpu.make_async_copy(q_hbm.at[didx.at[j]], qb[par], gsq[par]).wait()

        def wait_scatter(j, par):
            pltpu.make_async_copy(qb[par], s_sh.at[didx.at[j]], ssem[par]).wait()

        start_gather(0, 0)

        def pair(jj, _):
            for par in (0, 1):
                j = 2 * jj + par
                @pl.when(j >= 1)
                def _():
                    wait_scatter(j - 1, 1 - par)

                @pl.when(j + 1 < nch)
                def _():
                    start_gather(j + 1, 1 - par)

                wait_gather(j, par)

                def row(r, _):
                    for rr in (0, 1):
                        for c in range(_C // _L):
                            sl = pl.ds(c * _L, _L)
                            qb[par][2 * r + rr, sl] = jnp.maximum(
                                pb[par][2 * r + rr, sl] + qb[par][2 * r + rr, sl], 0.0)
                    return 0
                lax.fori_loop(0, _K // 2, row, 0)

                pltpu.async_copy(qb[par], s_sh.at[didx.at[j]], ssem[par],
                                 add=True)
            return 0
        lax.fori_loop(0, nch // 2, pair, 0)

        wait_scatter(nch - 1, 1)
        plsc.subcore_barrier()

        pltpu.sync_copy(s_sh.at[pl.ds(base, rpt)], s_out.at[cid, pl.ds(base, rpt)])

    return k(p, q, srcw, dstw)


def _sc_degree(dstw, n):
    nch = dstw.shape[1]
    rpt = n // _NS
    nz, zrem = rpt // _K, rpt % _K
    mesh = plsc.VectorSubcoreMesh(core_axis_name="c", subcore_axis_name="s")

    @functools.partial(
        pl.kernel,
        out_type=jax.ShapeDtypeStruct((_NC, n, _DPAD), jnp.float32),
        mesh=mesh,
        compiler_params=pltpu.CompilerParams(use_tc_tiling_on_sc=False),
        scratch_types=[
            pltpu.VMEM((nch, _K), jnp.int32),
            pltpu.VMEM((_K, _DPAD), jnp.float32),
            pltpu.VMEM((_K, _DPAD), jnp.float32),
            pltpu.VMEM_SHARED((n, _DPAD), jnp.float32),
            pltpu.SemaphoreType.DMA,
        ],
    )
    def k(dst_hbm, d_out, didx, obuf, zbuf, d_sh, dsem):
        cid = lax.axis_index("c")
        sid = lax.axis_index("s")
        wid = cid * _NS + sid
        pltpu.sync_copy(dst_hbm.at[wid], didx)

        ones = jnp.ones((_L,), jnp.float32)
        zeros = jnp.zeros((_L,), jnp.float32)

        def fill(r, _):
            obuf[r, :] = ones
            zbuf[r, :] = zeros
            return 0
        lax.fori_loop(0, _K, fill, 0)

        base = sid * rpt
        for t in range(nz):
            pltpu.sync_copy(zbuf, d_sh.at[pl.ds(base + t * _K, _K)])
        if zrem:
            pltpu.sync_copy(zbuf.at[pl.ds(0, zrem)],
                            d_sh.at[pl.ds(base + nz * _K, zrem)])
        plsc.subcore_barrier()

        def fire(j, _):
            pltpu.async_copy(obuf, d_sh.at[didx.at[j]], dsem, add=True)
            return 0
        lax.fori_loop(0, nch, fire, 0)

        def drain(j, _):
            pltpu.make_async_copy(obuf, d_sh.at[didx.at[0]], dsem).wait()
            return 0
        lax.fori_loop(0, nch, drain, 0)

        plsc.subcore_barrier()
        pltpu.sync_copy(d_sh.at[pl.ds(base, rpt)],
                        d_out.at[cid, pl.ds(base, rpt)])

    return k(dstw)

    return k(p, q, srcw, dstw)


_BR = 2000


def _row_spec(width):
    return pl.BlockSpec((_BR, width), lambda i: (i, 0))


def _w_spec(shape):
    return pl.BlockSpec(shape, lambda i: (0,) * len(shape))


def _enc_pq_body(x, pos, w1, b1, w2, b2, wp, wq, wd, bq, h_o, p_o, q_o):
    h = jnp.maximum(x[...] @ w1[...] + b1[...], 0.0) @ w2[...] + b2[...]
    pd = pos[...] @ wd[...]
    h_o[...] = h
    p_o[...] = h @ wp[...] + pd
    q_o[...] = h @ wq[...] - pd + bq[...]


def _enc_pq(x, pos8, w1t, b1, w2t, b2, wpt, wqt, wdt, bq):
    n = x.shape[0]
    grid = (n // _BR,)
    out = [jax.ShapeDtypeStruct((n, _C), jnp.float32)] * 3
    return pl.pallas_call(
        _enc_pq_body,
        grid=grid,
        in_specs=[
            _row_spec(_C), _row_spec(8),
            _w_spec((_C, _C)), _w_spec((1, _C)), _w_spec((_C, _C)), _w_spec((1, _C)),
            _w_spec((_C, _C)), _w_spec((_C, _C)), _w_spec((8, _C)), _w_spec((1, _C)),
        ],
        out_specs=[_row_spec(_C)] * 3,
        out_shape=out,
    )(x, pos8, w1t, b1, w2t, b2, wpt, wqt, wdt, bq)


def _node_core(h, s0, s1, deg, w2et, b2e, wn1at, wn1bt, bn1, wn2t, bn2):
    s = s0[...] + s1[...]
    agg = s @ w2et[...] + deg[...] * b2e[...]
    z = h[...] @ wn1at[...] + agg @ wn1bt[...] + bn1[...]
    return h[...] + jnp.maximum(z, 0.0) @ wn2t[...] + bn2[...]


def _node_pq_body(h, s0, s1, deg, pos,
                  w2et, b2e, wn1at, wn1bt, bn1, wn2t, bn2,
                  wp, wq, wd, bq, h_o, p_o, q_o):
    hn = _node_core(h, s0, s1, deg, w2et, b2e, wn1at, wn1bt, bn1, wn2t, bn2)
    pd = pos[...] @ wd[...]
    h_o[...] = hn
    p_o[...] = hn @ wp[...] + pd
    q_o[...] = hn @ wq[...] - pd + bq[...]


def _node_pq(h, s0, s1, deg, pos8, w2et, b2e, wn1at, wn1bt, bn1, wn2t, bn2,
             wpt, wqt, wdt, bq):
    n = h.shape[0]
    grid = (n // _BR,)
    out = [jax.ShapeDtypeStruct((n, _C), jnp.float32)] * 3
    return pl.pallas_call(
        _node_pq_body,
        grid=grid,
        in_specs=[
            _row_spec(_C), _row_spec(_C), _row_spec(_C), _row_spec(1), _row_spec(8),
            _w_spec((_C, _C)), _w_spec((1, _C)),
            _w_spec((_C, _C)), _w_spec((_C, _C)), _w_spec((1, _C)),
            _w_spec((_C, _C)), _w_spec((1, _C)),
            _w_spec((_C, _C)), _w_spec((_C, _C)), _w_spec((8, _C)), _w_spec((1, _C)),
        ],
        out_specs=[_row_spec(_C)] * 3,
        out_shape=out,
    )(h, s0, s1, deg, pos8, w2et, b2e, wn1at, wn1bt, bn1, wn2t, bn2,
      wpt, wqt, wdt, bq)


def _node_dec_body(h, s0, s1, deg,
                   w2et, b2e, wn1at, wn1bt, bn1, wn2t, bn2,
                   wd1, bd1, wd2, bd2, y_o):
    hn = _node_core(h, s0, s1, deg, w2et, b2e, wn1at, wn1bt, bn1, wn2t, bn2)
    y_o[...] = jnp.maximum(hn @ wd1[...] + bd1[...], 0.0) @ wd2[...] + bd2[...]


def _node_dec(h, s0, s1, deg, w2et, b2e, wn1at, wn1bt, bn1, wn2t, bn2,
              wd1t, bd1, wd2t, bd2):
    n = h.shape[0]
    grid = (n // _BR,)
    return pl.pallas_call(
        _node_dec_body,
        grid=grid,
        in_specs=[
            _row_spec(_C), _row_spec(_C), _row_spec(_C), _row_spec(1),
            _w_spec((_C, _C)), _w_spec((1, _C)),
            _w_spec((_C, _C)), _w_spec((_C, _C)), _w_spec((1, _C)),
            _w_spec((_C, _C)), _w_spec((1, _C)),
            _w_spec((_C, _C)), _w_spec((1, _C)), _w_spec((_C, _C)), _w_spec((1, _C)),
        ],
        out_specs=[_row_spec(_C)],
        out_shape=[jax.ShapeDtypeStruct((n, _C), jnp.float32)],
    )(h, s0, s1, deg, w2et, b2e, wn1at, wn1bt, bn1, wn2t, bn2,
      wd1t, bd1, wd2t, bd2)[0]


def _row(b):
    return b.reshape(1, -1)


@jax.jit
def kernel(x, edge_index, pos, params):
    n, c = x.shape
    e = edge_index.shape[1]
    epw = e // _NW
    nch = epw // _K
    assert nch * _K == epw and c == _C

    enc, dec, mps = params["enc"], params["dec"], params["mp"]
    pos8 = jnp.pad(pos, ((0, 0), (0, 5)))

    def edge_split(mp):
        w1 = mp["edge"][0]["W"]
        a, b, cm, d = w1[:, :c], w1[:, c:2*c], w1[:, 2*c:3*c], w1[:, 3*c:]
        wpt = (a + cm).T
        wqt = (b - cm).T
        wdt = jnp.pad(d, ((0, 0), (0, 5))).T
        return wpt, wqt, wdt, _row(mp["edge"][0]["b"])

    def node_split(mp):
        wn1 = mp["node"][0]["W"]
        return (mp["edge"][1]["W"].T, _row(mp["edge"][1]["b"]),
                wn1[:, :c].T, wn1[:, c:].T, _row(mp["node"][0]["b"]),
                mp["node"][1]["W"].T, _row(mp["node"][1]["b"]))

    src = edge_index[0].astype(jnp.int32).reshape(_NW, nch, _K)
    dst = edge_index[1].astype(jnp.int32).reshape(_NW, nch, _K)

    wpt, wqt, wdt, bq = edge_split(mps[0])
    h, p1, q1 = _enc_pq(x, pos8, enc[0]["W"].T, _row(enc[0]["b"]),
                        enc[1]["W"].T, _row(enc[1]["b"]), wpt, wqt, wdt, bq)

    d_part = _sc_degree(dst, n)
    s_part = _sc_edge_scatter(p1, q1, src, dst)
    deg = (d_part[0, :, 0] + d_part[1, :, 0]).reshape(n, 1)

    wpt2, wqt2, wdt2, bq2 = edge_split(mps[1])
    h2, p2, q2 = _node_pq(h, s_part[0], s_part[1], deg, pos8,
                          *node_split(mps[0]), wpt2, wqt2, wdt2, bq2)

    s_part2 = _sc_edge_scatter(p2, q2, src, dst)

    return _node_dec(h2, s_part2[0], s_part2[1], deg,
                     *node_split(mps[1]),
                     dec[0]["W"].T, _row(dec[0]["b"]),
                     dec[1]["W"].T, _row(dec[1]["b"]))
